# baseline (speedup 1.0000x reference)
import functools

import jax
import jax.numpy as jnp
from jax import lax
from jax.experimental import pallas as pl
from jax.experimental.pallas import tpu as pltpu

_CONTRACT_L = (((0,), (0,)), ((), ()))

_RING = 4


def _embed_kernel(x_hbm, xm_hbm, w_hbm, b_hbm, o_hbm,
                  xm_s, w_s, ym_ref, xbuf, obuf, sems, xsems, osems,
                  *, n, m, tb, l):
    g = pl.program_id(0)
    ng = pl.num_programs(0)
    nb = xm_s.shape[1] // m
    xslot = lax.rem(g, xbuf.shape[0])

    @pl.when(g == 0)
    def _():
        pltpu.make_async_copy(x_hbm.at[pl.ds(0, tb)],
                              xbuf.at[0, :, pl.ds(0, l), :],
                              xsems.at[0]).start()
        pltpu.make_async_copy(xm_hbm, xm_s.at[pl.ds(0, l), :],
                              sems.at[0]).start()
        pltpu.make_async_copy(w_hbm, w_s.at[pl.ds(0, l), :],
                              sems.at[1]).start()
        pltpu.make_async_copy(b_hbm, w_s.at[pl.ds(l, 1), :],
                              sems.at[2]).start()
        pltpu.make_async_copy(xm_hbm, xm_s.at[pl.ds(0, l), :],
                              sems.at[0]).wait()
        pltpu.make_async_copy(w_hbm, w_s.at[pl.ds(0, l), :],
                              sems.at[1]).wait()
        pltpu.make_async_copy(b_hbm, w_s.at[pl.ds(l, 1), :],
                              sems.at[2]).wait()
        xm_s[l, :] = jnp.ones((xm_s.shape[1],), jnp.float32)
        for s in range(xbuf.shape[0]):
            xbuf[s, :, l, :] = jnp.ones(
                (xbuf.shape[1], xbuf.shape[3]), jnp.float32)
        ym_ref[...] = lax.dot_general(
            xm_s[pl.ds(0, l + 1), :], w_s[pl.ds(0, l + 1), :], _CONTRACT_L,
            preferred_element_type=jnp.float32)

    @pl.when(g + 1 < ng)
    def _():
        nxt = lax.rem(g + 1, xbuf.shape[0])
        pltpu.make_async_copy(x_hbm.at[pl.ds((g + 1) * tb, tb)],
                              xbuf.at[nxt, :, pl.ds(0, l), :],
                              xsems.at[nxt]).start()

    pltpu.make_async_copy(x_hbm.at[pl.ds(0, tb)],
                          xbuf.at[xslot, :, pl.ds(0, l), :],
                          xsems.at[xslot]).wait()

    ring = obuf.shape[0]
    slot = lax.rem(g, ring)

    @pl.when(g >= ring)
    def _():
        pltpu.make_async_copy(obuf.at[slot], obuf.at[slot],
                              osems.at[slot]).wait()

    w = w_s[pl.ds(0, l + 1), :]
    for i in range(tb):
        y = lax.dot_general(xbuf[xslot, i, pl.ds(0, l + 1), :], w,
                            _CONTRACT_L,
                            preferred_element_type=jnp.float32)
        obuf[slot, :n, i, :] = y
    b0 = g * tb
    for j in range(m):
        obuf[slot, n + j, :, :] = ym_ref[pl.ds(j * nb + b0, tb), :]

    pltpu.make_async_copy(obuf.at[slot], o_hbm.at[:, pl.ds(b0, tb), :],
                          osems.at[slot]).start()

    @pl.when(g == ng - 1)
    def _():
        for s in range(ring):
            pltpu.make_async_copy(obuf.at[s], obuf.at[s], osems.at[s]).wait()


def kernel(x, x_mark, weight, bias):
    B, L, N = x.shape
    M = x_mark.shape[2]
    V = N + M
    D = weight.shape[0]

    w_t = jnp.transpose(weight, (1, 0))
    b2d = bias.reshape(1, D)
    xm2 = jnp.transpose(x_mark, (1, 2, 0)).reshape(L, M * B)

    tb = 8 if B % 8 == 0 else 1
    gb = B // tb

    ring = min(_RING, gb)
    out_t = pl.pallas_call(
        functools.partial(_embed_kernel, n=N, m=M, tb=tb, l=L),
        out_shape=jax.ShapeDtypeStruct((V, B, D), x.dtype),
        grid=(gb,),
        in_specs=[
            pl.BlockSpec(memory_space=pl.ANY),
            pl.BlockSpec(memory_space=pl.ANY),
            pl.BlockSpec(memory_space=pl.ANY),
            pl.BlockSpec(memory_space=pl.ANY),
        ],
        out_specs=pl.BlockSpec(memory_space=pl.ANY),
        scratch_shapes=[
            pltpu.VMEM((L + 1, M * B), jnp.float32),
            pltpu.VMEM((L + 1, D), jnp.float32),
            pltpu.VMEM((M * B, D), jnp.float32),
            pltpu.VMEM((2, tb, L + 1, N), jnp.float32),
            pltpu.VMEM((ring, V, tb, D), jnp.float32),
            pltpu.SemaphoreType.DMA((3,)),
            pltpu.SemaphoreType.DMA((2,)),
            pltpu.SemaphoreType.DMA((ring,)),
        ],
        compiler_params=pltpu.CompilerParams(
            dimension_semantics=("arbitrary",),
            vmem_limit_bytes=56 * 1024 * 1024,
        ),
        cost_estimate=pl.CostEstimate(
            flops=2 * B * V * L * D,
            transcendentals=0,
            bytes_accessed=4 * (B * L * V + B * V * D) + 2 * L * D + 4 * D,
        ),
    )(x, xm2, w_t, b2d)
    return jnp.transpose(out_t, (1, 0, 2))


# speedup vs baseline: 1.0015x; 1.0015x over previous
import functools

import jax
import jax.numpy as jnp
from jax import lax
from jax.experimental import pallas as pl
from jax.experimental.pallas import tpu as pltpu

_CONTRACT_L = (((0,), (0,)), ((), ()))

_RING = 3


def _embed_kernel(x_hbm, xm_hbm, w_hbm, b_hbm, o_hbm,
                  xm_s, w_s, ym_ref, xbuf, obuf, sems, xsems, osems,
                  *, n, m, tb, l):
    g = pl.program_id(0)
    ng = pl.num_programs(0)
    nb = xm_s.shape[1] // m
    xslot = lax.rem(g, xbuf.shape[0])

    @pl.when(g == 0)
    def _():
        pltpu.make_async_copy(x_hbm.at[pl.ds(0, tb)],
                              xbuf.at[0, :, pl.ds(0, l), :],
                              xsems.at[0]).start()
        pltpu.make_async_copy(xm_hbm, xm_s.at[pl.ds(0, l), :],
                              sems.at[0]).start()
        pltpu.make_async_copy(w_hbm, w_s.at[pl.ds(0, l), :],
                              sems.at[1]).start()
        pltpu.make_async_copy(b_hbm, w_s.at[pl.ds(l, 1), :],
                              sems.at[2]).start()
        pltpu.make_async_copy(xm_hbm, xm_s.at[pl.ds(0, l), :],
                              sems.at[0]).wait()
        pltpu.make_async_copy(w_hbm, w_s.at[pl.ds(0, l), :],
                              sems.at[1]).wait()
        pltpu.make_async_copy(b_hbm, w_s.at[pl.ds(l, 1), :],
                              sems.at[2]).wait()
        xm_s[l, :] = jnp.ones((xm_s.shape[1],), jnp.float32)
        for s in range(xbuf.shape[0]):
            xbuf[s, :, l, :] = jnp.ones(
                (xbuf.shape[1], xbuf.shape[3]), jnp.float32)
        ym_ref[...] = lax.dot_general(
            xm_s[pl.ds(0, l + 1), :], w_s[pl.ds(0, l + 1), :], _CONTRACT_L,
            preferred_element_type=jnp.float32)

    @pl.when(g + 1 < ng)
    def _():
        nxt = lax.rem(g + 1, xbuf.shape[0])
        pltpu.make_async_copy(x_hbm.at[pl.ds((g + 1) * tb, tb)],
                              xbuf.at[nxt, :, pl.ds(0, l), :],
                              xsems.at[nxt]).start()

    pltpu.make_async_copy(x_hbm.at[pl.ds(0, tb)],
                          xbuf.at[xslot, :, pl.ds(0, l), :],
                          xsems.at[xslot]).wait()

    ring = obuf.shape[0]
    slot = lax.rem(g, ring)

    @pl.when(g >= ring)
    def _():
        pltpu.make_async_copy(obuf.at[slot], obuf.at[slot],
                              osems.at[slot]).wait()

    w = w_s[pl.ds(0, l + 1), :]
    for i in range(tb):
        y = lax.dot_general(xbuf[xslot, i, pl.ds(0, l + 1), :], w,
                            _CONTRACT_L,
                            preferred_element_type=jnp.float32)
        obuf[slot, :n, i, :] = y
    b0 = g * tb
    for j in range(m):
        obuf[slot, n + j, :, :] = ym_ref[pl.ds(j * nb + b0, tb), :]

    pltpu.make_async_copy(obuf.at[slot], o_hbm.at[:, pl.ds(b0, tb), :],
                          osems.at[slot]).start()

    @pl.when(g == ng - 1)
    def _():
        for s in range(ring):
            pltpu.make_async_copy(obuf.at[s], obuf.at[s], osems.at[s]).wait()


def kernel(x, x_mark, weight, bias):
    B, L, N = x.shape
    M = x_mark.shape[2]
    V = N + M
    D = weight.shape[0]

    w_t = jnp.transpose(weight, (1, 0))
    b2d = bias.reshape(1, D)
    xm2 = jnp.transpose(x_mark, (1, 2, 0)).reshape(L, M * B)

    tb = 8 if B % 8 == 0 else 1
    gb = B // tb

    ring = min(_RING, gb)
    out_t = pl.pallas_call(
        functools.partial(_embed_kernel, n=N, m=M, tb=tb, l=L),
        out_shape=jax.ShapeDtypeStruct((V, B, D), x.dtype),
        grid=(gb,),
        in_specs=[
            pl.BlockSpec(memory_space=pl.ANY),
            pl.BlockSpec(memory_space=pl.ANY),
            pl.BlockSpec(memory_space=pl.ANY),
            pl.BlockSpec(memory_space=pl.ANY),
        ],
        out_specs=pl.BlockSpec(memory_space=pl.ANY),
        scratch_shapes=[
            pltpu.VMEM((L + 1, M * B), jnp.float32),
            pltpu.VMEM((L + 1, D), jnp.float32),
            pltpu.VMEM((M * B, D), jnp.float32),
            pltpu.VMEM((2, tb, L + 1, N), jnp.float32),
            pltpu.VMEM((ring, V, tb, D), jnp.float32),
            pltpu.SemaphoreType.DMA((3,)),
            pltpu.SemaphoreType.DMA((2,)),
            pltpu.SemaphoreType.DMA((ring,)),
        ],
        compiler_params=pltpu.CompilerParams(
            dimension_semantics=("arbitrary",),
            vmem_limit_bytes=56 * 1024 * 1024,
        ),
        cost_estimate=pl.CostEstimate(
            flops=2 * B * V * L * D,
            transcendentals=0,
            bytes_accessed=4 * (B * L * V + B * V * D) + 2 * L * D + 4 * D,
        ),
    )(x, xm2, w_t, b2d)
    return jnp.transpose(out_t, (1, 0, 2))
